# Initial kernel scaffold; baseline (speedup 1.0000x reference)
import jax
import jax.numpy as jnp
from jax.experimental import pallas as pl


def kernel(x, edge_index, edge_attr, batch, atom_origin_type, params):
    raise NotImplementedError("write your pallas kernel here")


# speedup vs baseline: 1.0810x; 1.0810x over previous
import functools

import jax
import jax.numpy as jnp
from jax.experimental import pallas as pl
from jax.experimental.pallas import tpu as pltpu

_N = 50000
_E = 800000
_DIN = 128
_DE = 16
_HID = 64
_H = 4
_C = 16
_DEPTH = 3
_G = 256


def _mm_relu_body(x_ref, w_ref, b_ref, o_ref):
    o_ref[...] = jax.nn.relu(
        jnp.dot(x_ref[...], w_ref[...], preferred_element_type=jnp.float32)
        + b_ref[...]
    )


def _mm_body(x_ref, w_ref, o_ref):
    o_ref[...] = jnp.dot(x_ref[...], w_ref[...], preferred_element_type=jnp.float32)


def _matmul(x, w, b=None, relu=False, bn=400):
    n, k = x.shape
    m = w.shape[1]
    grid = (n // bn,)
    in_specs = [
        pl.BlockSpec((bn, k), lambda i: (i, 0)),
        pl.BlockSpec((k, m), lambda i: (0, 0)),
    ]
    args = [x, w]
    if relu:
        in_specs.append(pl.BlockSpec((1, m), lambda i: (0, 0)))
        args.append(b.reshape(1, m))
        body = _mm_relu_body
    else:
        body = _mm_body
    return pl.pallas_call(
        body,
        grid=grid,
        in_specs=in_specs,
        out_specs=pl.BlockSpec((bn, m), lambda i: (i, 0)),
        out_shape=jax.ShapeDtypeStruct((n, m), jnp.float32),
    )(*args)


def kernel(x, edge_index, edge_attr, batch, atom_origin_type, params):
    src, dst = edge_index[0], edge_index[1]
    cnt = jnp.zeros((_N,), x.dtype).at[dst].add(1.0)
    sums = jnp.zeros((_N, _DE), x.dtype).at[dst].add(edge_attr)
    loop_attr = sums / jnp.maximum(cnt, 1.0)[:, None]
    ar = jnp.arange(_N, dtype=src.dtype)
    src2 = jnp.concatenate([src, ar])
    dst2 = jnp.concatenate([dst, ar])
    ea2 = jnp.concatenate([edge_attr, loop_attr], axis=0)

    h0 = _matmul(x, params['W_init'], params['b_init'], relu=True)
    h = h0
    for l in range(_DEPTH):
        xs = _matmul(h, params[f'W{l}']).reshape(_N, _H, _C)
        a_src = (xs * params[f'att_src{l}'][None]).sum(-1)
        a_dst = (xs * params[f'att_dst{l}'][None]).sum(-1)
        me = jnp.einsum('dhc,hc->dh', params[f'We{l}'].reshape(_DE, _H, _C),
                        params[f'att_e{l}'])
        a_e = ea2 @ me
        alpha = a_src[src2] + a_dst[dst2] + a_e
        alpha = jax.nn.leaky_relu(alpha, 0.2)
        ex = jnp.exp(alpha)
        den = jnp.zeros((_N, _H), x.dtype).at[dst2].add(ex)
        msg = xs[src2] * ex[..., None]
        num = jnp.zeros((_N, _H, _C), x.dtype).at[dst2].add(msg)
        h = (num / (den[..., None] + 1e-16)).reshape(_N, _HID) + params[f'bias{l}']
        h = jax.nn.relu(h + h0)
    pooled = jnp.zeros((_G, _HID), x.dtype).at[batch].add(h)
    z = jax.nn.relu(pooled @ params['W_f1'] + params['b_f1'])
    out = (z @ params['W_f2'] + params['b_f2']).squeeze(-1)
    return out
